# baseline (speedup 1.0000x reference)
import functools

import jax
import jax.numpy as jnp
from jax import lax
from jax.experimental import pallas as pl
from jax.experimental.pallas import tpu as pltpu
from jax.experimental.pallas import tpu_sc as plsc

DIM = 768
HID = 2048
E = 8
K = 2
T = 2048
MT = 128
TP = K * T + E * MT
NT = TP // MT
NTO = 48

NC = 2
NS = 16
NW = NC * NS
GB = TP // NW
GCH = GB // 2
GC4 = GB // 4
TW = T // NW
NL = DIM // 16


def _gate_body(x_ref, wg_ref, i1_ref, i2_ref, w1_ref, w2_ref):
    xb = x_ref[...]
    logits = lax.dot_general(xb, wg_ref[...], (((1,), (1,)), ((), ())),
                             preferred_element_type=jnp.float32)
    m = jnp.max(logits, axis=-1, keepdims=True)
    p = jnp.exp(logits - m)
    p = p / jnp.sum(p, axis=-1, keepdims=True)
    iota = lax.broadcasted_iota(jnp.int32, (T, E), 1)
    m1 = jnp.max(p, axis=-1, keepdims=True)
    i1 = jnp.min(jnp.where(p == m1, iota, E), axis=-1, keepdims=True)
    sel1 = iota == i1
    p2 = jnp.where(sel1, -jnp.inf, p)
    m2 = jnp.max(p2, axis=-1, keepdims=True)
    i2 = jnp.min(jnp.where(p2 == m2, iota, E), axis=-1, keepdims=True)
    denom = m1 + m2 + 1e-20
    i1_ref[...] = i1
    i2_ref[...] = i2
    w1_ref[...] = m1 / denom
    w2_ref[...] = m2 / denom


def _gate(flat, Wg):
    return pl.pallas_call(
        _gate_body,
        grid=(1,),
        in_specs=[pl.BlockSpec((T, DIM), lambda i: (0, 0)),
                  pl.BlockSpec((E, DIM), lambda i: (0, 0))],
        out_specs=[pl.BlockSpec((T, 1), lambda i: (0, 0))] * 4,
        out_shape=[jax.ShapeDtypeStruct((T, 1), jnp.int32),
                   jax.ShapeDtypeStruct((T, 1), jnp.int32),
                   jax.ShapeDtypeStruct((T, 1), jnp.float32),
                   jax.ShapeDtypeStruct((T, 1), jnp.float32)],
    )(flat, Wg)


SLW = 2 * T // NS
STW = TP // NS


@functools.partial(
    pl.kernel,
    mesh=plsc.VectorSubcoreMesh(core_axis_name="c", subcore_axis_name="s"),
    out_type=[jax.ShapeDtypeStruct((TP,), jnp.int32),
              jax.ShapeDtypeStruct((TP,), jnp.float32),
              jax.ShapeDtypeStruct((NTO,), jnp.int32),
              jax.ShapeDtypeStruct((2 * T,), jnp.int32)],
    scratch_types=[pltpu.VMEM((SLW,), jnp.int32),
                   pltpu.VMEM((SLW,), jnp.float32),
                   pltpu.VMEM((SLW,), jnp.int32),
                   pltpu.VMEM((SLW,), jnp.int32),
                   pltpu.VMEM((16,), jnp.int32),
                   pltpu.VMEM((16 * NS,), jnp.int32),
                   pltpu.VMEM((NTO,), jnp.int32),
                   pltpu.VMEM((STW,), jnp.int32),
                   pltpu.VMEM((STW,), jnp.float32),
                   pltpu.SMEM((16,), jnp.int32),
                   pltpu.SMEM((NTO,), jnp.int32),
                   pltpu.VMEM_SHARED((16 * NS,), jnp.int32),
                   pltpu.VMEM_SHARED((TP,), jnp.int32),
                   pltpu.VMEM_SHARED((TP,), jnp.float32)],
)
def _sc_route(i1_hbm, i2_hbm, wa_hbm, wb_hbm,
              stid_hbm, sw_hbm, teid_hbm, dest_hbm,
              ev, wv, tokv, destv, cntv, allv, teidv, zbi, zbf,
              cnt_s, teid_s, cnt_sh, stid_sh, sw_sh):
    cid = lax.axis_index("c")
    sid = lax.axis_index("s")

    @pl.when(cid == 0)
    def _():
        base = sid * SLW
        tok_base = jnp.where(sid < 8, base, base - T)
        lanes = lax.iota(jnp.int32, 16)

        @pl.when(sid < 8)
        def _load_k0():
            pltpu.sync_copy(i1_hbm.at[pl.ds(base, SLW)], ev)
            pltpu.sync_copy(wa_hbm.at[pl.ds(base, SLW)], wv)

        @pl.when(sid >= 8)
        def _load_k1():
            pltpu.sync_copy(i2_hbm.at[pl.ds(base - T, SLW)], ev)
            pltpu.sync_copy(wb_hbm.at[pl.ds(base - T, SLW)], wv)

        for e in range(16):
            cnt_s[e] = 0

        def hist(c, _):
            chunk = ev[pl.ds(c * 16, 16)]
            for j in range(16):
                e = chunk[j]
                cnt_s[e] = cnt_s[e] + 1
            return 0
        lax.fori_loop(0, SLW // 16, hist, 0)

        acc = jnp.zeros((16,), jnp.int32)
        for j in range(16):
            acc = jnp.where(lanes == j, jnp.full((16,), cnt_s[j], jnp.int32),
                            acc)
        cntv[...] = acc

        def zz(i, _):
            zbi[pl.ds(i * 16, 16)] = jnp.zeros((16,), jnp.int32)
            zbf[pl.ds(i * 16, 16)] = jnp.zeros((16,), jnp.float32)
            return 0
        lax.fori_loop(0, STW // 16, zz, 0)
        pltpu.sync_copy(cntv, cnt_sh.at[pl.ds(sid * 16, 16)])
        pltpu.sync_copy(zbi, stid_sh.at[pl.ds(sid * STW, STW)])
        pltpu.sync_copy(zbf, sw_sh.at[pl.ds(sid * STW, STW)])
        plsc.subcore_barrier()

        pltpu.sync_copy(cnt_sh, allv)

        def vsum(lo, hi):
            def body(w2, a):
                return a + allv[pl.ds(w2 * 16, 16)]
            return lax.fori_loop(lo, hi, body, jnp.zeros((16,), jnp.int32))
        totv = vsum(0, NS)
        minev = vsum(0, sid)
        off = 0
        for e in range(E):
            tot_e = totv[e]
            start_e = off + minev[e]
            cnt_s[e] = start_e
            off = off + lax.shift_left(
                lax.shift_right_logical(tot_e + (MT - 1), 7), 7)

        @pl.when(sid == 0)
        def _teid():
            t = 0
            for e in range(E):
                nt_e = lax.shift_right_logical(totv[e] + (MT - 1), 7)

                def fill(j, _, e=e, t=t):
                    teid_s[t + j] = e
                    return 0
                lax.fori_loop(0, nt_e, fill, 0)
                t = t + nt_e

            def fill7(j, _):
                teid_s[j] = E - 1
                return 0
            lax.fori_loop(t, NTO, fill7, 0)
            teid_s[40] = t
            for c in range(NTO // 16):
                acc2 = jnp.zeros((16,), jnp.int32)
                for j in range(16):
                    acc2 = jnp.where(
                        lanes == j,
                        jnp.full((16,), teid_s[c * 16 + j], jnp.int32), acc2)
                teidv[pl.ds(c * 16, 16)] = acc2
            pltpu.sync_copy(teidv, teid_hbm)

        def place(c, _):
            chunk = ev[pl.ds(c * 16, 16)]
            dacc = jnp.zeros((16,), jnp.int32)
            for j in range(16):
                e = chunk[j]
                d = cnt_s[e]
                cnt_s[e] = d + 1
                dacc = jnp.where(lanes == j, jnp.full((16,), d, jnp.int32),
                                 dacc)
            destv[pl.ds(c * 16, 16)] = dacc
            tokv[pl.ds(c * 16, 16)] = lanes + jnp.full(
                (16,), tok_base + c * 16, jnp.int32)
            return 0
        lax.fori_loop(0, SLW // 16, place, 0)

        pltpu.sync_copy(tokv, stid_sh.at[destv])
        pltpu.sync_copy(wv, sw_sh.at[destv])
        pltpu.sync_copy(destv, dest_hbm.at[pl.ds(base, SLW)])
        plsc.subcore_barrier()

        pltpu.sync_copy(stid_sh.at[pl.ds(sid * STW, STW)], zbi)
        pltpu.sync_copy(zbi, stid_hbm.at[pl.ds(sid * STW, STW)])
        pltpu.sync_copy(sw_sh.at[pl.ds(sid * STW, STW)], zbf)
        pltpu.sync_copy(zbf, sw_hbm.at[pl.ds(sid * STW, STW)])


@functools.partial(
    pl.kernel,
    mesh=plsc.VectorSubcoreMesh(core_axis_name="c", subcore_axis_name="s"),
    out_type=jax.ShapeDtypeStruct((TP, DIM), jnp.float32),
    scratch_types=[pltpu.VMEM((GB,), jnp.int32),
                   pltpu.VMEM((GB, DIM), jnp.float32),
                   pltpu.SemaphoreType.DMA, pltpu.SemaphoreType.DMA,
                   pltpu.SemaphoreType.DMA, pltpu.SemaphoreType.DMA],
)
def _sc_gather(x_hbm, idx_hbm, out_hbm, idx_v, rows, sa, sb, sc_, sd):
    wid = lax.axis_index("s") * NC + lax.axis_index("c")
    base = wid * GB
    pltpu.sync_copy(idx_hbm.at[pl.ds(base, GB)], idx_v)
    gsems = [sa, sb, sc_, sd]
    gs = [pltpu.async_copy(x_hbm.at[idx_v.at[pl.ds(i * GC4, GC4)]],
                           rows.at[pl.ds(i * GC4, GC4)], gsems[i])
          for i in range(4)]
    for g in gs:
        g.wait()
    pltpu.sync_copy(rows, out_hbm.at[pl.ds(base, GB)])


def _ffn_body(eid_ref, x_ref, w1_ref, w3_ref, w2_ref, sw_ref, o_ref):
    t = pl.program_id(0)

    @pl.when(t < eid_ref[40])
    def _():
        xb = x_ref[...]
        h1 = lax.dot_general(xb, w1_ref[0], (((1,), (1,)), ((), ())),
                             preferred_element_type=jnp.float32)
        h3 = lax.dot_general(xb, w3_ref[0], (((1,), (1,)), ((), ())),
                             preferred_element_type=jnp.float32)
        hid = (h1 * jax.nn.sigmoid(h1)) * h3
        out = lax.dot_general(hid, w2_ref[0], (((1,), (1,)), ((), ())),
                              preferred_element_type=jnp.float32)
        o_ref[...] = sw_ref[...] * out


def _grouped_ffn(xg, W1, W3, W2, sorted_w, tile_eid):
    def tmap(t, eid):
        return (jnp.minimum(t, eid[40] - 1), 0)

    def wmap(t, eid):
        return (eid[jnp.minimum(t, eid[40] - 1)], 0, 0)

    grid_spec = pltpu.PrefetchScalarGridSpec(
        num_scalar_prefetch=1,
        grid=(NT,),
        in_specs=[
            pl.BlockSpec((MT, DIM), tmap),
            pl.BlockSpec((1, HID, DIM), wmap),
            pl.BlockSpec((1, HID, DIM), wmap),
            pl.BlockSpec((1, DIM, HID), wmap),
            pl.BlockSpec((MT, 1), tmap),
        ],
        out_specs=pl.BlockSpec((MT, DIM), lambda t, eid: (t, 0)),
    )
    return pl.pallas_call(
        _ffn_body,
        grid_spec=grid_spec,
        out_shape=jax.ShapeDtypeStruct((TP, DIM), jnp.float32),
    )(tile_eid, xg, W1, W3, W2, sorted_w[:, None])


@functools.partial(
    pl.kernel,
    mesh=plsc.VectorSubcoreMesh(core_axis_name="c", subcore_axis_name="s"),
    out_type=jax.ShapeDtypeStruct((T, DIM), jnp.float32),
    scratch_types=[pltpu.VMEM((TW,), jnp.int32),
                   pltpu.VMEM((TW,), jnp.int32),
                   pltpu.VMEM((TW, DIM), jnp.float32),
                   pltpu.VMEM((TW, DIM), jnp.float32),
                   pltpu.SemaphoreType.DMA, pltpu.SemaphoreType.DMA],
)
def _sc_combine(os_hbm, dest_hbm, y_hbm, idx1_v, idx2_v, rows1, rows2, sa, sb):
    wid = lax.axis_index("s") * NC + lax.axis_index("c")
    tok = wid * TW
    pltpu.sync_copy(dest_hbm.at[pl.ds(tok, TW)], idx1_v)
    pltpu.sync_copy(dest_hbm.at[pl.ds(T + tok, TW)], idx2_v)
    g1 = pltpu.async_copy(os_hbm.at[idx1_v], rows1, sa)
    g2 = pltpu.async_copy(os_hbm.at[idx2_v], rows2, sb)
    g1.wait()
    g2.wait()

    def rbody(t_loc, _):
        def lbody(c, _):
            sl = pl.ds(c * 16, 16)
            rows1[t_loc, sl] = rows1[t_loc, sl] + rows2[t_loc, sl]
            return 0
        return lax.fori_loop(0, NL, lbody, 0)

    lax.fori_loop(0, TW, rbody, 0)
    pltpu.sync_copy(rows1, y_hbm.at[pl.ds(tok, TW)])


def kernel(x, Wg, W1, W2, W3):
    b, s, d = x.shape
    flat = x.reshape(T, d)
    i1, i2, w1n, w2n = _gate(flat, Wg)
    stid, sw, teid, dest = _sc_route(i1[:, 0], i2[:, 0], w1n[:, 0], w2n[:, 0])
    xg = _sc_gather(flat, stid)
    os_ = _grouped_ffn(xg, W1, W3, W2, sw, teid)
    y = _sc_combine(os_, dest)
    return y.reshape(b, s, d)


# speedup vs baseline: 1.0693x; 1.0693x over previous
import functools

import jax
import jax.numpy as jnp
from jax import lax
from jax.experimental import pallas as pl
from jax.experimental.pallas import tpu as pltpu
from jax.experimental.pallas import tpu_sc as plsc

DIM = 768
HID = 2048
E = 8
K = 2
T = 2048
MT = 128
TP = K * T + E * MT
NT = TP // MT
NTO = 48

NC = 2
NS = 16
NW = NC * NS
GB = TP // NW
GCH = GB // 2
GC4 = GB // 4
TW = T // NW
NL = DIM // 16


def _gate_body(x_ref, wg_ref, i1_ref, i2_ref, w1_ref, w2_ref):
    xb = x_ref[...]
    logits = lax.dot_general(xb, wg_ref[...], (((1,), (1,)), ((), ())),
                             preferred_element_type=jnp.float32)
    m = jnp.max(logits, axis=-1, keepdims=True)
    p = jnp.exp(logits - m)
    p = p / jnp.sum(p, axis=-1, keepdims=True)
    iota = lax.broadcasted_iota(jnp.int32, (T, E), 1)
    m1 = jnp.max(p, axis=-1, keepdims=True)
    i1 = jnp.min(jnp.where(p == m1, iota, E), axis=-1, keepdims=True)
    sel1 = iota == i1
    p2 = jnp.where(sel1, -jnp.inf, p)
    m2 = jnp.max(p2, axis=-1, keepdims=True)
    i2 = jnp.min(jnp.where(p2 == m2, iota, E), axis=-1, keepdims=True)
    denom = m1 + m2 + 1e-20
    i1_ref[...] = i1
    i2_ref[...] = i2
    w1_ref[...] = m1 / denom
    w2_ref[...] = m2 / denom


def _gate(flat, Wg):
    return pl.pallas_call(
        _gate_body,
        grid=(1,),
        in_specs=[pl.BlockSpec((T, DIM), lambda i: (0, 0)),
                  pl.BlockSpec((E, DIM), lambda i: (0, 0))],
        out_specs=[pl.BlockSpec((T, 1), lambda i: (0, 0))] * 4,
        out_shape=[jax.ShapeDtypeStruct((T, 1), jnp.int32),
                   jax.ShapeDtypeStruct((T, 1), jnp.int32),
                   jax.ShapeDtypeStruct((T, 1), jnp.float32),
                   jax.ShapeDtypeStruct((T, 1), jnp.float32)],
    )(flat, Wg)


SLW = 2 * T // NS
STW = TP // NS


@functools.partial(
    pl.kernel,
    mesh=plsc.VectorSubcoreMesh(core_axis_name="c", subcore_axis_name="s"),
    out_type=[jax.ShapeDtypeStruct((TP,), jnp.int32),
              jax.ShapeDtypeStruct((TP,), jnp.float32),
              jax.ShapeDtypeStruct((NTO,), jnp.int32),
              jax.ShapeDtypeStruct((2 * T,), jnp.int32)],
    scratch_types=[pltpu.VMEM((SLW,), jnp.int32),
                   pltpu.VMEM((SLW,), jnp.float32),
                   pltpu.VMEM((SLW,), jnp.int32),
                   pltpu.VMEM((SLW,), jnp.int32),
                   pltpu.VMEM((16,), jnp.int32),
                   pltpu.VMEM((16 * NS,), jnp.int32),
                   pltpu.VMEM((NTO,), jnp.int32),
                   pltpu.VMEM((STW,), jnp.int32),
                   pltpu.VMEM((STW,), jnp.float32),
                   pltpu.SMEM((16,), jnp.int32),
                   pltpu.SMEM((NTO,), jnp.int32),
                   pltpu.VMEM_SHARED((16 * NS,), jnp.int32),
                   pltpu.VMEM_SHARED((TP,), jnp.int32),
                   pltpu.VMEM_SHARED((TP,), jnp.float32)],
)
def _sc_route(i1_hbm, i2_hbm, wa_hbm, wb_hbm,
              stid_hbm, sw_hbm, teid_hbm, dest_hbm,
              ev, wv, tokv, destv, cntv, allv, teidv, zbi, zbf,
              cnt_s, teid_s, cnt_sh, stid_sh, sw_sh):
    cid = lax.axis_index("c")
    sid = lax.axis_index("s")

    @pl.when(cid == 0)
    def _():
        base = sid * SLW
        tok_base = jnp.where(sid < 8, base, base - T)
        lanes = lax.iota(jnp.int32, 16)

        @pl.when(sid < 8)
        def _load_k0():
            pltpu.sync_copy(i1_hbm.at[pl.ds(base, SLW)], ev)
            pltpu.sync_copy(wa_hbm.at[pl.ds(base, SLW)], wv)

        @pl.when(sid >= 8)
        def _load_k1():
            pltpu.sync_copy(i2_hbm.at[pl.ds(base - T, SLW)], ev)
            pltpu.sync_copy(wb_hbm.at[pl.ds(base - T, SLW)], wv)

        for e in range(16):
            cnt_s[e] = 0

        def hist(c, _):
            chunk = ev[pl.ds(c * 16, 16)]
            for j in range(16):
                e = chunk[j]
                cnt_s[e] = cnt_s[e] + 1
            return 0
        lax.fori_loop(0, SLW // 16, hist, 0)

        acc = jnp.zeros((16,), jnp.int32)
        for j in range(16):
            acc = jnp.where(lanes == j, jnp.full((16,), cnt_s[j], jnp.int32),
                            acc)
        cntv[...] = acc

        def zz(i, _):
            zbi[pl.ds(i * 16, 16)] = jnp.zeros((16,), jnp.int32)
            zbf[pl.ds(i * 16, 16)] = jnp.zeros((16,), jnp.float32)
            return 0
        lax.fori_loop(0, STW // 16, zz, 0)
        pltpu.sync_copy(cntv, cnt_sh.at[pl.ds(sid * 16, 16)])
        pltpu.sync_copy(zbi, stid_sh.at[pl.ds(sid * STW, STW)])
        pltpu.sync_copy(zbf, sw_sh.at[pl.ds(sid * STW, STW)])
        plsc.subcore_barrier()

        pltpu.sync_copy(cnt_sh, allv)

        def vsum(lo, hi):
            def body(w2, a):
                return a + allv[pl.ds(w2 * 16, 16)]
            return lax.fori_loop(lo, hi, body, jnp.zeros((16,), jnp.int32))
        totv = vsum(0, NS)
        minev = vsum(0, sid)
        off = 0
        for e in range(E):
            tot_e = totv[e]
            start_e = off + minev[e]
            cnt_s[e] = start_e
            off = off + lax.shift_left(
                lax.shift_right_logical(tot_e + (MT - 1), 7), 7)

        @pl.when(sid == 0)
        def _teid():
            t = 0
            for e in range(E):
                nt_e = lax.shift_right_logical(totv[e] + (MT - 1), 7)

                def fill(j, _, e=e, t=t):
                    teid_s[t + j] = e
                    return 0
                lax.fori_loop(0, nt_e, fill, 0)
                t = t + nt_e

            def fill7(j, _):
                teid_s[j] = E - 1
                return 0
            lax.fori_loop(t, NTO, fill7, 0)
            teid_s[40] = t
            for c in range(NTO // 16):
                acc2 = jnp.zeros((16,), jnp.int32)
                for j in range(16):
                    acc2 = jnp.where(
                        lanes == j,
                        jnp.full((16,), teid_s[c * 16 + j], jnp.int32), acc2)
                teidv[pl.ds(c * 16, 16)] = acc2
            pltpu.sync_copy(teidv, teid_hbm)

        def place(c, _):
            chunk = ev[pl.ds(c * 16, 16)]
            dacc = jnp.zeros((16,), jnp.int32)
            for j in range(16):
                e = chunk[j]
                d = cnt_s[e]
                cnt_s[e] = d + 1
                dacc = jnp.where(lanes == j, jnp.full((16,), d, jnp.int32),
                                 dacc)
            destv[pl.ds(c * 16, 16)] = dacc
            tokv[pl.ds(c * 16, 16)] = lanes + jnp.full(
                (16,), tok_base + c * 16, jnp.int32)
            return 0
        lax.fori_loop(0, SLW // 16, place, 0)

        pltpu.sync_copy(tokv, stid_sh.at[destv])
        pltpu.sync_copy(wv, sw_sh.at[destv])
        pltpu.sync_copy(destv, dest_hbm.at[pl.ds(base, SLW)])
        plsc.subcore_barrier()

        pltpu.sync_copy(stid_sh.at[pl.ds(sid * STW, STW)], zbi)
        pltpu.sync_copy(zbi, stid_hbm.at[pl.ds(sid * STW, STW)])
        pltpu.sync_copy(sw_sh.at[pl.ds(sid * STW, STW)], zbf)
        pltpu.sync_copy(zbf, sw_hbm.at[pl.ds(sid * STW, STW)])


@functools.partial(
    pl.kernel,
    mesh=plsc.VectorSubcoreMesh(core_axis_name="c", subcore_axis_name="s"),
    out_type=jax.ShapeDtypeStruct((TP, DIM), jnp.float32),
    scratch_types=[pltpu.VMEM((GB,), jnp.int32),
                   pltpu.VMEM((NTO,), jnp.int32),
                   pltpu.VMEM((GB, DIM), jnp.float32),
                   pltpu.SemaphoreType.DMA],
)
def _sc_gather(x_hbm, idx_hbm, teid_hbm, out_hbm, idx_v, teidv, rows, sa):
    wid = lax.axis_index("s") * NC + lax.axis_index("c")
    base = wid * GB
    pltpu.sync_copy(teid_hbm, teidv)
    used = teidv[pl.ds(32, 16)][8] * MT
    pltpu.sync_copy(idx_hbm.at[pl.ds(base, GB)], idx_v)
    for i in range(4):
        @pl.when(base + i * GC4 < used)
        def _g(i=i):
            pltpu.async_copy(x_hbm.at[idx_v.at[pl.ds(i * GC4, GC4)]],
                             rows.at[pl.ds(i * GC4, GC4)], sa).wait()
    pltpu.sync_copy(rows, out_hbm.at[pl.ds(base, GB)])


def _ffn_body(eid_ref, x_ref, w1_ref, w3_ref, w2_ref, sw_ref, o_ref):
    t = pl.program_id(0)

    @pl.when(t < eid_ref[40])
    def _():
        xb = x_ref[...]
        h1 = lax.dot_general(xb, w1_ref[0], (((1,), (1,)), ((), ())),
                             preferred_element_type=jnp.float32)
        h3 = lax.dot_general(xb, w3_ref[0], (((1,), (1,)), ((), ())),
                             preferred_element_type=jnp.float32)
        hid = (h1 * jax.nn.sigmoid(h1)) * h3
        out = lax.dot_general(hid, w2_ref[0], (((1,), (1,)), ((), ())),
                              preferred_element_type=jnp.float32)
        o_ref[...] = sw_ref[...] * out


def _grouped_ffn(xg, W1, W3, W2, sorted_w, tile_eid):
    def tmap(t, eid):
        return (jnp.minimum(t, eid[40] - 1), 0)

    def wmap(t, eid):
        return (eid[jnp.minimum(t, eid[40] - 1)], 0, 0)

    grid_spec = pltpu.PrefetchScalarGridSpec(
        num_scalar_prefetch=1,
        grid=(NT,),
        in_specs=[
            pl.BlockSpec((MT, DIM), tmap),
            pl.BlockSpec((1, HID, DIM), wmap),
            pl.BlockSpec((1, HID, DIM), wmap),
            pl.BlockSpec((1, DIM, HID), wmap),
            pl.BlockSpec((MT, 1), tmap),
        ],
        out_specs=pl.BlockSpec((MT, DIM), lambda t, eid: (t, 0)),
    )
    return pl.pallas_call(
        _ffn_body,
        grid_spec=grid_spec,
        out_shape=jax.ShapeDtypeStruct((TP, DIM), jnp.float32),
    )(tile_eid, xg, W1, W3, W2, sorted_w[:, None])


@functools.partial(
    pl.kernel,
    mesh=plsc.VectorSubcoreMesh(core_axis_name="c", subcore_axis_name="s"),
    out_type=jax.ShapeDtypeStruct((T, DIM), jnp.float32),
    scratch_types=[pltpu.VMEM((TW,), jnp.int32),
                   pltpu.VMEM((TW,), jnp.int32),
                   pltpu.VMEM((TW, DIM), jnp.float32),
                   pltpu.VMEM((TW, DIM), jnp.float32),
                   pltpu.SemaphoreType.DMA, pltpu.SemaphoreType.DMA],
)
def _sc_combine(os_hbm, dest_hbm, y_hbm, idx1_v, idx2_v, rows1, rows2, sa, sb):
    wid = lax.axis_index("s") * NC + lax.axis_index("c")
    tok = wid * TW
    pltpu.sync_copy(dest_hbm.at[pl.ds(tok, TW)], idx1_v)
    pltpu.sync_copy(dest_hbm.at[pl.ds(T + tok, TW)], idx2_v)
    g1 = pltpu.async_copy(os_hbm.at[idx1_v], rows1, sa)
    g2 = pltpu.async_copy(os_hbm.at[idx2_v], rows2, sb)
    g1.wait()
    g2.wait()

    def rbody(t_loc, _):
        def lbody(c, _):
            for u in range(4):
                sl = pl.ds((c * 4 + u) * 16, 16)
                rows1[t_loc, sl] = rows1[t_loc, sl] + rows2[t_loc, sl]
            return 0
        return lax.fori_loop(0, NL // 4, lbody, 0)

    lax.fori_loop(0, TW, rbody, 0)
    pltpu.sync_copy(rows1, y_hbm.at[pl.ds(tok, TW)])


def kernel(x, Wg, W1, W2, W3):
    b, s, d = x.shape
    flat = x.reshape(T, d)
    i1, i2, w1n, w2n = _gate(flat, Wg)
    stid, sw, teid, dest = _sc_route(i1[:, 0], i2[:, 0], w1n[:, 0], w2n[:, 0])
    xg = _sc_gather(flat, stid, teid)
    os_ = _grouped_ffn(xg, W1, W3, W2, sw, teid)
    y = _sc_combine(os_, dest)
    return y.reshape(b, s, d)
